# baseline (speedup 1.0000x reference)
import jax
import jax.numpy as jnp
from jax import lax
from jax.experimental import pallas as pl
from jax.experimental.pallas import tpu as pltpu
from jax.experimental.pallas import tpu_sc as plsc

N = 10000
E = 160000
D = 256
NC = 2
NS = 16
HN = N // NC
CH = 48
TCAP_CH = 240
TCAP = TCAP_CH * CH
PHASES = 6
PCH = TCAP_CH // PHASES
TRASH = HN
ACC_ROWS = HN + 16
IR = 312
IR_LAST = HN - 15 * IR


def _agg_body(h_hbm, src_hbm, dst_hbm, cnt_hbm, out_hbm,
              src_v, dst_v, cnt_v, cnt_s, buf0, buf1, acc, sem0, sem1):
    c = lax.axis_index("c")
    s = lax.axis_index("s")
    del cnt_hbm, cnt_v, cnt_s

    @pl.when(s < 15)
    def _():
        pltpu.sync_copy(h_hbm.at[pl.ds(c * HN + s * IR, IR)],
                        acc.at[pl.ds(s * IR, IR)])

    @pl.when(s == 15)
    def _():
        pltpu.sync_copy(h_hbm.at[pl.ds(c * HN + 15 * IR, IR_LAST)],
                        acc.at[pl.ds(15 * IR, IR_LAST)])

    plsc.subcore_barrier()

    n = TCAP_CH

    for p in range(PHASES):
        np_ = n - p * PCH

        @pl.when(np_ > 0)
        def _():
            npc = jnp.minimum(np_, PCH)
            pltpu.sync_copy(src_hbm.at[c, s, pl.ds(p * PCH, PCH)], src_v)
            pltpu.sync_copy(dst_hbm.at[c, s, pl.ds(p * PCH, PCH)], dst_v)
            pltpu.async_copy(h_hbm.at[src_v.at[0]], buf0, sem0)
            pltpu.async_copy(h_hbm.at[src_v.at[1]], buf1, sem1)

            @pl.loop(0, npc, step=2)
            def _(j):
                pltpu.make_async_copy(h_hbm.at[src_v.at[j]], buf0, sem0).wait()
                pltpu.sync_copy(buf0, acc.at[dst_v.at[j]], add=True)

                @pl.when(j + 2 < npc)
                def _():
                    pltpu.async_copy(h_hbm.at[src_v.at[j + 2]], buf0, sem0)

                pltpu.make_async_copy(h_hbm.at[src_v.at[j + 1]], buf1,
                                      sem1).wait()
                pltpu.sync_copy(buf1, acc.at[dst_v.at[j + 1]], add=True)

                @pl.when(j + 3 < npc)
                def _():
                    pltpu.async_copy(h_hbm.at[src_v.at[j + 3]], buf1, sem1)

    plsc.subcore_barrier()

    @pl.when(s < 15)
    def _():
        pltpu.sync_copy(acc.at[pl.ds(s * IR, IR)],
                        out_hbm.at[pl.ds(c * HN + s * IR, IR)])

    @pl.when(s == 15)
    def _():
        pltpu.sync_copy(acc.at[pl.ds(15 * IR, IR_LAST)],
                        out_hbm.at[pl.ds(c * HN + 15 * IR, IR_LAST)])


@jax.jit
def _sc_agg(h, src_i, dst_i, cnt):
    mesh = plsc.VectorSubcoreMesh(core_axis_name="c", subcore_axis_name="s")
    k = pl.kernel(
        _agg_body,
        out_type=jax.ShapeDtypeStruct((N, 2, 128), jnp.float32),
        mesh=mesh,
        scratch_types=[
            pltpu.VMEM((PCH, CH), jnp.int32),
            pltpu.VMEM((PCH, CH), jnp.int32),
            pltpu.VMEM((16,), jnp.int32),
            pltpu.SMEM((16,), jnp.int32),
            pltpu.VMEM((CH, 2, 128), jnp.float32),
            pltpu.VMEM((CH, 2, 128), jnp.float32),
            pltpu.VMEM_SHARED((ACC_ROWS, 2, 128), jnp.float32),
            pltpu.SemaphoreType.DMA,
            pltpu.SemaphoreType.DMA,
        ],
    )
    return k(h.reshape(N, 2, 128), src_i, dst_i, cnt).reshape(N, D)


RB = 2000
NBLK = N // RB
_W = pl.BlockSpec((D, D), lambda i: (0, 0))
_B = pl.BlockSpec((1, D), lambda i: (0, 0))
_R = pl.BlockSpec((RB, D), lambda i: (i, 0))


def _mm1_body(z_ref, w1_ref, b1_ref, h_ref, s_ref, q_ref):
    h = jnp.dot(z_ref[...], w1_ref[...],
                preferred_element_type=jnp.float32) + b1_ref[:]
    h_ref[...] = h

    @pl.when(pl.program_id(0) == 0)
    def _():
        s_ref[...] = jnp.zeros_like(s_ref)
        q_ref[...] = jnp.zeros_like(q_ref)

    s_ref[...] += jnp.sum(h, axis=0, keepdims=True)
    q_ref[...] += jnp.sum(h * h, axis=0, keepdims=True)


def _mm2_body(h_ref, s_ref, q_ref, g_ref, be_ref, w2_ref, b2_ref, o_ref):
    m = s_ref[...] * (1.0 / N)
    v = q_ref[...] * (1.0 / N) - m * m
    hn = (h_ref[...] - m) * lax.rsqrt(v + 1e-5) * g_ref[:] + be_ref[:]
    hn = jnp.maximum(hn, 0.0)
    o_ref[...] = jnp.maximum(
        jnp.dot(hn, w2_ref[...], preferred_element_type=jnp.float32)
        + b2_ref[:], 0.0)


def _mlp01(z, w1, b1, g, be, w2, b2):
    h, s, q = pl.pallas_call(
        _mm1_body,
        grid=(NBLK,),
        in_specs=[_R, _W, _B],
        out_specs=[_R, _B, _B],
        out_shape=[
            jax.ShapeDtypeStruct((N, D), jnp.float32),
            jax.ShapeDtypeStruct((1, D), jnp.float32),
            jax.ShapeDtypeStruct((1, D), jnp.float32),
        ],
    )(z, w1, b1)
    return pl.pallas_call(
        _mm2_body,
        grid=(NBLK,),
        in_specs=[_R, _B, _B, _B, _B, _W, _B],
        out_specs=_R,
        out_shape=jax.ShapeDtypeStruct((N, D), jnp.float32),
    )(h, s, q, g, be, w2, b2)


def _mlp2_body(z_ref, w1_ref, b1_ref, w2_ref, b2_ref, o_ref):
    h = jnp.dot(z_ref[...], w1_ref[...],
                preferred_element_type=jnp.float32) + b1_ref[:]
    o_ref[...] = jnp.dot(h, w2_ref[...],
                         preferred_element_type=jnp.float32) + b2_ref[:]


def _mlp2(z, w1, b1, w2, b2):
    return pl.pallas_call(
        _mlp2_body,
        grid=(NBLK,),
        in_specs=[_R, _W, _B, _W, _B],
        out_specs=_R,
        out_shape=jax.ShapeDtypeStruct((N, D), jnp.float32),
    )(z, w1, b1, w2, b2)


def kernel(x, edge_index, l0_W1, l0_b1, l0_W2, l0_b2, l0_g, l0_beta,
           l1_W1, l1_b1, l1_W2, l1_b2, l1_g, l1_beta,
           l2_W1, l2_b1, l2_W2, l2_b2):
    src = edge_index[0].astype(jnp.int32)
    dst = edge_index[1].astype(jnp.int32)

    key = (dst >= HN).astype(jnp.int32)
    n1 = jnp.sum(key)
    n0 = E - n1
    c1 = jnp.cumsum(key)
    pos = jnp.where(key == 1, c1 - 1, jnp.arange(E, dtype=jnp.int32) - c1)
    tile = pos % NS
    slot = key * (NS * TCAP) + tile * TCAP + pos // NS
    dstl = dst - HN * key
    src_flat = jnp.zeros((NC * NS * TCAP,), jnp.int32).at[slot].set(src)
    dst_flat = jnp.full((NC * NS * TCAP,), TRASH, jnp.int32).at[slot].set(dstl)
    src_i = src_flat.reshape(NC, NS, TCAP_CH, CH)
    dst_i = dst_flat.reshape(NC, NS, TCAP_CH, CH)

    t = jnp.arange(NS, dtype=jnp.int32)
    nct = (jnp.stack([n0, n1])[:, None] - t[None, :] + NS - 1) // NS
    nch = (nct + CH - 1) // CH
    nch = jnp.maximum(((nch + 1) // 2) * 2, 2)
    cnt = jnp.broadcast_to(nch[:, :, None], (NC, NS, 16)).astype(jnp.int32)

    b = lambda a: a.reshape(1, D)

    z = _sc_agg(x, src_i, dst_i, cnt)
    h = _mlp01(z, l0_W1, b(l0_b1), b(l0_g), b(l0_beta), l0_W2, b(l0_b2))
    z = _sc_agg(h, src_i, dst_i, cnt)
    h = _mlp01(z, l1_W1, b(l1_b1), b(l1_g), b(l1_beta), l1_W2, b(l1_b2))
    z = _sc_agg(h, src_i, dst_i, cnt)
    return _mlp2(z, l2_W1, b(l2_b1), l2_W2, b(l2_b2))


# speedup vs baseline: 12.7414x; 12.7414x over previous
import dataclasses

import jax
import jax.numpy as jnp
from jax import lax
from jax.experimental import pallas as pl
from jax.experimental.pallas import tpu as pltpu
from jax.experimental.pallas import tpu_sc as plsc

N = 10000
E = 160000
D = 256
NC = 2
NS = 16
HN = N // NC
CH = 48
TCAP_CH = 240
TCAP = TCAP_CH * CH
PHASES = 6
PCH = TCAP_CH // PHASES
TRASH = HN
ACC_ROWS = HN + 16
IR = 312
IR_LAST = HN - 15 * IR


def _agg_body(h_hbm, src_hbm, dst_hbm, cnt_hbm, out_hbm,
              src_v, dst_v, cnt_v, cnt_s, buf0, buf1, acc, sem0, sem1):
    c = lax.axis_index("c")
    s = lax.axis_index("s")
    pltpu.sync_copy(cnt_hbm.at[c, s], cnt_v)
    del cnt_s

    @pl.when(s < 15)
    def _():
        pltpu.sync_copy(h_hbm.at[pl.ds(c * HN + s * IR, IR)],
                        acc.at[pl.ds(s * IR, IR)])

    @pl.when(s == 15)
    def _():
        pltpu.sync_copy(h_hbm.at[pl.ds(c * HN + 15 * IR, IR_LAST)],
                        acc.at[pl.ds(15 * IR, IR_LAST)])

    plsc.subcore_barrier()

    n = jnp.max(cnt_v[...])

    for p in range(PHASES):
        np_ = n - p * PCH

        @pl.when(np_ > 0)
        def _():
            npc = jnp.minimum(np_, PCH)
            pltpu.sync_copy(src_hbm.at[c, s, pl.ds(p * PCH, PCH)], src_v)
            pltpu.sync_copy(dst_hbm.at[c, s, pl.ds(p * PCH, PCH)], dst_v)
            pltpu.async_copy(h_hbm.at[src_v.at[0]], buf0, sem0)
            pltpu.async_copy(h_hbm.at[src_v.at[1]], buf1, sem1)

            @pl.loop(0, npc, step=2)
            def _(j):
                pltpu.make_async_copy(h_hbm.at[src_v.at[j]], buf0, sem0).wait()
                pltpu.sync_copy(buf0, acc.at[dst_v.at[j]], add=True)

                @pl.when(j + 2 < npc)
                def _():
                    pltpu.async_copy(h_hbm.at[src_v.at[j + 2]], buf0, sem0)

                pltpu.make_async_copy(h_hbm.at[src_v.at[j + 1]], buf1,
                                      sem1).wait()
                pltpu.sync_copy(buf1, acc.at[dst_v.at[j + 1]], add=True)

                @pl.when(j + 3 < npc)
                def _():
                    pltpu.async_copy(h_hbm.at[src_v.at[j + 3]], buf1, sem1)

    plsc.subcore_barrier()

    @pl.when(s < 15)
    def _():
        pltpu.sync_copy(acc.at[pl.ds(s * IR, IR)],
                        out_hbm.at[pl.ds(c * HN + s * IR, IR)])

    @pl.when(s == 15)
    def _():
        pltpu.sync_copy(acc.at[pl.ds(15 * IR, IR_LAST)],
                        out_hbm.at[pl.ds(c * HN + 15 * IR, IR_LAST)])


@jax.jit
def _sc_agg(h, src_i, dst_i, cnt):
    mesh = plsc.VectorSubcoreMesh(core_axis_name="c", subcore_axis_name="s")
    cp = pltpu.CompilerParams()
    if "needs_layout_passes" in pltpu.CompilerParams.__dataclass_fields__:
        cp = dataclasses.replace(cp, needs_layout_passes=False)
    k = pl.kernel(
        _agg_body,
        out_type=jax.ShapeDtypeStruct((N, 2, 128), jnp.float32),
        mesh=mesh,
        compiler_params=cp,
        scratch_types=[
            pltpu.VMEM((PCH, CH), jnp.int32),
            pltpu.VMEM((PCH, CH), jnp.int32),
            pltpu.VMEM((16,), jnp.int32),
            pltpu.SMEM((16,), jnp.int32),
            pltpu.VMEM((CH, 2, 128), jnp.float32),
            pltpu.VMEM((CH, 2, 128), jnp.float32),
            pltpu.VMEM_SHARED((ACC_ROWS, 2, 128), jnp.float32),
            pltpu.SemaphoreType.DMA,
            pltpu.SemaphoreType.DMA,
        ],
    )
    return k(h.reshape(N, 2, 128), src_i, dst_i, cnt).reshape(N, D)


RB = 2000
NBLK = N // RB
_W = pl.BlockSpec((D, D), lambda i: (0, 0))
_B = pl.BlockSpec((1, D), lambda i: (0, 0))
_R = pl.BlockSpec((RB, D), lambda i: (i, 0))


def _mm1_body(z_ref, w1_ref, b1_ref, h_ref, s_ref, q_ref):
    h = jnp.dot(z_ref[...], w1_ref[...],
                preferred_element_type=jnp.float32) + b1_ref[:]
    h_ref[...] = h

    @pl.when(pl.program_id(0) == 0)
    def _():
        s_ref[...] = jnp.zeros_like(s_ref)
        q_ref[...] = jnp.zeros_like(q_ref)

    s_ref[...] += jnp.sum(h, axis=0, keepdims=True)
    q_ref[...] += jnp.sum(h * h, axis=0, keepdims=True)


def _mm2_body(h_ref, s_ref, q_ref, g_ref, be_ref, w2_ref, b2_ref, o_ref):
    m = s_ref[...] * (1.0 / N)
    v = q_ref[...] * (1.0 / N) - m * m
    hn = (h_ref[...] - m) * lax.rsqrt(v + 1e-5) * g_ref[:] + be_ref[:]
    hn = jnp.maximum(hn, 0.0)
    o_ref[...] = jnp.maximum(
        jnp.dot(hn, w2_ref[...], preferred_element_type=jnp.float32)
        + b2_ref[:], 0.0)


def _mlp01(z, w1, b1, g, be, w2, b2):
    h, s, q = pl.pallas_call(
        _mm1_body,
        grid=(NBLK,),
        in_specs=[_R, _W, _B],
        out_specs=[_R, _B, _B],
        out_shape=[
            jax.ShapeDtypeStruct((N, D), jnp.float32),
            jax.ShapeDtypeStruct((1, D), jnp.float32),
            jax.ShapeDtypeStruct((1, D), jnp.float32),
        ],
    )(z, w1, b1)
    return pl.pallas_call(
        _mm2_body,
        grid=(NBLK,),
        in_specs=[_R, _B, _B, _B, _B, _W, _B],
        out_specs=_R,
        out_shape=jax.ShapeDtypeStruct((N, D), jnp.float32),
    )(h, s, q, g, be, w2, b2)


def _mlp2_body(z_ref, w1_ref, b1_ref, w2_ref, b2_ref, o_ref):
    h = jnp.dot(z_ref[...], w1_ref[...],
                preferred_element_type=jnp.float32) + b1_ref[:]
    o_ref[...] = jnp.dot(h, w2_ref[...],
                         preferred_element_type=jnp.float32) + b2_ref[:]


def _mlp2(z, w1, b1, w2, b2):
    return pl.pallas_call(
        _mlp2_body,
        grid=(NBLK,),
        in_specs=[_R, _W, _B, _W, _B],
        out_specs=_R,
        out_shape=jax.ShapeDtypeStruct((N, D), jnp.float32),
    )(z, w1, b1, w2, b2)


def kernel(x, edge_index, l0_W1, l0_b1, l0_W2, l0_b2, l0_g, l0_beta,
           l1_W1, l1_b1, l1_W2, l1_b2, l1_g, l1_beta,
           l2_W1, l2_b1, l2_W2, l2_b2):
    src = edge_index[0].astype(jnp.int32)
    dst = edge_index[1].astype(jnp.int32)

    key = (dst >= HN).astype(jnp.int32)
    n1 = jnp.sum(key)
    n0 = E - n1
    c1 = jnp.cumsum(key)
    pos = jnp.where(key == 1, c1 - 1, jnp.arange(E, dtype=jnp.int32) - c1)
    tile = pos % NS
    slot = key * (NS * TCAP) + tile * TCAP + pos // NS
    dstl = dst - HN * key
    src_flat = jnp.zeros((NC * NS * TCAP,), jnp.int32).at[slot].set(src)
    trash = TRASH + jnp.arange(NC * NS * TCAP, dtype=jnp.int32) % 8
    dst_flat = trash.at[slot].set(dstl)
    src_i = src_flat.reshape(NC, NS, TCAP_CH, CH)
    dst_i = dst_flat.reshape(NC, NS, TCAP_CH, CH)

    t = jnp.arange(NS, dtype=jnp.int32)
    nct = (jnp.stack([n0, n1])[:, None] - t[None, :] + NS - 1) // NS
    nch = (nct + CH - 1) // CH
    nch = jnp.maximum(((nch + 1) // 2) * 2, 2)
    cnt = jnp.broadcast_to(nch[:, :, None], (NC, NS, 16)).astype(jnp.int32)

    b = lambda a: a.reshape(1, D)

    z = _sc_agg(x, src_i, dst_i, cnt)
    h = _mlp01(z, l0_W1, b(l0_b1), b(l0_g), b(l0_beta), l0_W2, b(l0_b2))
    z = _sc_agg(h, src_i, dst_i, cnt)
    h = _mlp01(z, l1_W1, b(l1_b1), b(l1_g), b(l1_beta), l1_W2, b(l1_b2))
    z = _sc_agg(h, src_i, dst_i, cnt)
    return _mlp2(z, l2_W1, b(l2_b1), l2_W2, b(l2_b2))


# speedup vs baseline: 30.2806x; 2.3765x over previous
import functools

import jax
import jax.numpy as jnp
from jax import lax
from jax.experimental import pallas as pl
from jax.experimental.pallas import tpu as pltpu
from jax.experimental.pallas import tpu_sc as plsc

N = 10000
E = 160000
D = 256
HALF = 128
NC = 2
NS = 16
EDGES_PER_TILE = E // NS
CHUNK = 128
NCHUNK = 10240 // CHUNK
PAD_PER_TILE = 10240 - EDGES_PER_TILE
PHASES = 2
PCH = NCHUNK // PHASES
ROWS_PER_TILE = 624
LAST_TILE_ROWS = N - 15 * ROWS_PER_TILE
ACC_ROWS = N + 16


def _agg_body(h_hbm, src_hbm, dst_hbm, out_hbm, src_v, dst_v, buf0, buf1,
              acc, sem0, sem1, ssem0, ssem1):
    c = lax.axis_index("c")
    s = lax.axis_index("s")
    @pl.when(s < 15)
    def _():
        pltpu.sync_copy(
            h_hbm.at[pl.ds(c * N + s * ROWS_PER_TILE, ROWS_PER_TILE)],
            acc.at[pl.ds(s * ROWS_PER_TILE, ROWS_PER_TILE)],
        )

    @pl.when(s == 15)
    def _():
        pltpu.sync_copy(
            h_hbm.at[pl.ds(c * N + 15 * ROWS_PER_TILE, LAST_TILE_ROWS)],
            acc.at[pl.ds(15 * ROWS_PER_TILE, LAST_TILE_ROWS)],
        )

    plsc.subcore_barrier()

    for p in range(PHASES):
        pltpu.sync_copy(src_hbm.at[c, s, pl.ds(p * PCH, PCH)], src_v)
        pltpu.sync_copy(dst_hbm.at[s, pl.ds(p * PCH, PCH)], dst_v)
        pltpu.async_copy(h_hbm.at[src_v.at[0]], buf0, sem0)
        pltpu.async_copy(h_hbm.at[src_v.at[1]], buf1, sem1)

        @pl.loop(0, PCH, step=2)
        def _(j):
            pltpu.make_async_copy(h_hbm.at[src_v.at[j]], buf0, sem0).wait()
            pltpu.sync_copy(buf0, acc.at[dst_v.at[j]], add=True)

            @pl.when(j + 2 < PCH)
            def _():
                pltpu.async_copy(h_hbm.at[src_v.at[j + 2]], buf0, sem0)

            pltpu.make_async_copy(h_hbm.at[src_v.at[j + 1]], buf1, sem1).wait()
            pltpu.sync_copy(buf1, acc.at[dst_v.at[j + 1]], add=True)

            @pl.when(j + 3 < PCH)
            def _():
                pltpu.async_copy(h_hbm.at[src_v.at[j + 3]], buf1, sem1)

    plsc.subcore_barrier()

    @pl.when(s < 15)
    def _():
        pltpu.sync_copy(
            acc.at[pl.ds(s * ROWS_PER_TILE, ROWS_PER_TILE)],
            out_hbm.at[pl.ds(c * N + s * ROWS_PER_TILE, ROWS_PER_TILE)],
        )

    @pl.when(s == 15)
    def _():
        pltpu.sync_copy(
            acc.at[pl.ds(15 * ROWS_PER_TILE, LAST_TILE_ROWS)],
            out_hbm.at[pl.ds(c * N + 15 * ROWS_PER_TILE, LAST_TILE_ROWS)],
        )


@jax.jit
def _sc_agg(h2, src_idx, dst_idx):
    mesh = plsc.VectorSubcoreMesh(core_axis_name="c", subcore_axis_name="s")
    k = pl.kernel(
        _agg_body,
        out_type=jax.ShapeDtypeStruct((NC * N, HALF), jnp.float32),
        mesh=mesh,
        scratch_types=[
            pltpu.VMEM((PCH, CHUNK), jnp.int32),
            pltpu.VMEM((PCH, CHUNK), jnp.int32),
            pltpu.VMEM((CHUNK, HALF), jnp.float32),
            pltpu.VMEM((CHUNK, HALF), jnp.float32),
            pltpu.VMEM_SHARED((ACC_ROWS, HALF), jnp.float32),
            pltpu.SemaphoreType.DMA,
            pltpu.SemaphoreType.DMA,
            pltpu.SemaphoreType.DMA,
            pltpu.SemaphoreType.DMA,
        ],
    )
    return k(h2, src_idx, dst_idx)


RB = 2000
NBLK = N // RB


def _mm1_body(z_ref, w1_ref, b1_ref, h_ref, s_ref, q_ref):
    hi = jax.lax.Precision.DEFAULT
    h = (
        jnp.dot(z_ref[0], w1_ref[:HALF, :], precision=hi,
                preferred_element_type=jnp.float32)
        + jnp.dot(z_ref[1], w1_ref[HALF:, :], precision=hi,
                  preferred_element_type=jnp.float32)
        + b1_ref[:]
    )
    h_ref[...] = h

    @pl.when(pl.program_id(0) == 0)
    def _():
        s_ref[...] = jnp.zeros_like(s_ref)
        q_ref[...] = jnp.zeros_like(q_ref)

    s_ref[...] += jnp.sum(h, axis=0, keepdims=True)
    q_ref[...] += jnp.sum(h * h, axis=0, keepdims=True)


def _mm2_body(h_ref, s_ref, q_ref, g_ref, be_ref, w2_ref, b2_ref, o_ref):
    hi = jax.lax.Precision.DEFAULT
    m = s_ref[...] * (1.0 / N)
    v = q_ref[...] * (1.0 / N) - m * m
    hn = (h_ref[...] - m) * jax.lax.rsqrt(v + 1e-5) * g_ref[:] + be_ref[:]
    hn = jnp.maximum(hn, 0.0)
    o_ref[0] = jnp.maximum(
        jnp.dot(hn, w2_ref[:, :HALF], precision=hi,
                preferred_element_type=jnp.float32) + b2_ref[:, :HALF], 0.0)
    o_ref[1] = jnp.maximum(
        jnp.dot(hn, w2_ref[:, HALF:], precision=hi,
                preferred_element_type=jnp.float32) + b2_ref[:, HALF:], 0.0)


def _mlp01(z2, w1, b1, g, be, w2, b2):
    h, s, q = pl.pallas_call(
        _mm1_body,
        grid=(NBLK,),
        in_specs=[
            pl.BlockSpec((2, RB, HALF), lambda i: (0, i, 0)),
            pl.BlockSpec((D, D), lambda i: (0, 0)),
            pl.BlockSpec((1, D), lambda i: (0, 0)),
        ],
        out_specs=[
            pl.BlockSpec((RB, D), lambda i: (i, 0)),
            pl.BlockSpec((1, D), lambda i: (0, 0)),
            pl.BlockSpec((1, D), lambda i: (0, 0)),
        ],
        out_shape=[
            jax.ShapeDtypeStruct((N, D), jnp.float32),
            jax.ShapeDtypeStruct((1, D), jnp.float32),
            jax.ShapeDtypeStruct((1, D), jnp.float32),
        ],
    )(z2.reshape(2, N, HALF), w1, b1)
    return pl.pallas_call(
        _mm2_body,
        grid=(NBLK,),
        in_specs=[
            pl.BlockSpec((RB, D), lambda i: (i, 0)),
            pl.BlockSpec((1, D), lambda i: (0, 0)),
            pl.BlockSpec((1, D), lambda i: (0, 0)),
            pl.BlockSpec((1, D), lambda i: (0, 0)),
            pl.BlockSpec((1, D), lambda i: (0, 0)),
            pl.BlockSpec((D, D), lambda i: (0, 0)),
            pl.BlockSpec((1, D), lambda i: (0, 0)),
        ],
        out_specs=pl.BlockSpec((2, RB, HALF), lambda i: (0, i, 0)),
        out_shape=jax.ShapeDtypeStruct((2, N, HALF), jnp.float32),
    )(h, s, q, g, be, w2, b2)


def _mlp2_body(z_ref, w1_ref, b1_ref, w2_ref, b2_ref, o_ref):
    hi = jax.lax.Precision.DEFAULT
    h = (
        jnp.dot(z_ref[0], w1_ref[:HALF, :], precision=hi,
                preferred_element_type=jnp.float32)
        + jnp.dot(z_ref[1], w1_ref[HALF:, :], precision=hi,
                  preferred_element_type=jnp.float32)
        + b1_ref[:]
    )
    o_ref[...] = (
        jnp.dot(h, w2_ref[...], precision=hi,
                preferred_element_type=jnp.float32) + b2_ref[:]
    )


def _mlp2(z2, w1, b1, w2, b2):
    return pl.pallas_call(
        _mlp2_body,
        grid=(NBLK,),
        in_specs=[
            pl.BlockSpec((2, RB, HALF), lambda i: (0, i, 0)),
            pl.BlockSpec((D, D), lambda i: (0, 0)),
            pl.BlockSpec((1, D), lambda i: (0, 0)),
            pl.BlockSpec((D, D), lambda i: (0, 0)),
            pl.BlockSpec((1, D), lambda i: (0, 0)),
        ],
        out_specs=pl.BlockSpec((RB, D), lambda i: (i, 0)),
        out_shape=jax.ShapeDtypeStruct((N, D), jnp.float32),
    )(z2.reshape(2, N, HALF), w1, b1, w2, b2)


def kernel(x, edge_index, l0_W1, l0_b1, l0_W2, l0_b2, l0_g, l0_beta,
           l1_W1, l1_b1, l1_W2, l1_b2, l1_g, l1_beta,
           l2_W1, l2_b1, l2_W2, l2_b2):
    src = edge_index[0].astype(jnp.int32)
    dst = edge_index[1].astype(jnp.int32)

    src_t = jnp.pad(src.reshape(NS, EDGES_PER_TILE), ((0, 0), (0, PAD_PER_TILE)))
    dst_t = jnp.pad(dst.reshape(NS, EDGES_PER_TILE), ((0, 0), (0, PAD_PER_TILE)),
                    constant_values=N)
    src_idx = jnp.stack([src_t, src_t + N]).reshape(NC, NS, NCHUNK, CHUNK)
    dst_idx = dst_t.reshape(NS, NCHUNK, CHUNK)

    x2 = x.reshape(N, 2, HALF).transpose(1, 0, 2).reshape(2 * N, HALF)

    b = lambda a: a.reshape(1, D)

    z = _sc_agg(x2, src_idx, dst_idx)
    h = _mlp01(z, l0_W1, b(l0_b1), b(l0_g), b(l0_beta), l0_W2, b(l0_b2))
    z = _sc_agg(h.reshape(2 * N, HALF), src_idx, dst_idx)
    h = _mlp01(z, l1_W1, b(l1_b1), b(l1_g), b(l1_beta), l1_W2, b(l1_b2))
    z = _sc_agg(h.reshape(2 * N, HALF), src_idx, dst_idx)
    return _mlp2(z, l2_W1, b(l2_b1), l2_W2, b(l2_b2))
